# baseline (speedup 1.0000x reference)
import functools

import jax
import jax.numpy as jnp
from jax import lax
from jax.experimental import pallas as pl
from jax.experimental.pallas import tpu as pltpu
from jax.experimental.pallas import tpu_sc as plsc

H = 128
HE = 16
NA = 1000
NE = 320000

NWORK = 32
CH = 1024
NCHUNK = 10
EPW = CH * NCHUNK
NEP = NWORK * EPW
NB = CH // 128
SROW = 1024
SPAD = 1 << 20
ZB = 8192


def _pre_body(a_ref, nf_ref, wq_ref, bq_ref, wkn_ref, wke_ref, wvn_ref,
              t1s_ref, qe_ref, nv_ref):
    q = jnp.dot(a_ref[...], wq_ref[...], preferred_element_type=jnp.float32,
                precision=lax.Precision.HIGHEST)
    q = q + bq_ref[...]
    nk = jnp.dot(nf_ref[...], wkn_ref[...], preferred_element_type=jnp.float32,
                precision=lax.Precision.HIGHEST)
    nv_ref[...] = jnp.dot(nf_ref[...], wvn_ref[...],
                          preferred_element_type=jnp.float32,
                precision=lax.Precision.HIGHEST)
    t1 = lax.dot_general(q, nk, (((1,), (1,)), ((), ())),
                         preferred_element_type=jnp.float32,
                precision=lax.Precision.HIGHEST)
    t1s_ref[...] = t1 - jnp.max(t1, axis=1, keepdims=True)
    qe_ref[...] = lax.dot_general(q, wke_ref[...], (((1,), (1,)), ((), ())),
                                  preferred_element_type=jnp.float32,
                precision=lax.Precision.HIGHEST)


def _pre(a, nf1, wq, bq, wkn, wke, wvn):
    return pl.pallas_call(
        _pre_body,
        out_shape=(
            jax.ShapeDtypeStruct((NA, SROW), jnp.float32),
            jax.ShapeDtypeStruct((NA, HE), jnp.float32),
            jax.ShapeDtypeStruct((SROW, H), jnp.float32),
        ),
    )(a, nf1, wq, bq, wkn, wke, wvn)


EFBLK = 1600

def _efr_body(ef_ref, out_ref):
    x = ef_ref[...]
    u = lax.bitcast_convert_type(x, jnp.uint32)
    lsb = (u >> jnp.uint32(16)) & jnp.uint32(1)
    u = (u + jnp.uint32(0x7FFF) + lsb) & jnp.uint32(0xFFFF0000)
    y = lax.bitcast_convert_type(u, jnp.float32)
    out_ref[...] = y.reshape(EFBLK * HE // 128, 128)


def _efr(ef):
    nblk = NE // EFBLK
    return pl.pallas_call(
        _efr_body,
        grid=(nblk,),
        in_specs=[pl.BlockSpec((EFBLK, HE), lambda i: (i, 0))],
        out_specs=pl.BlockSpec((EFBLK * HE // 128, 128), lambda i: (i, 0)),
        out_shape=jax.ShapeDtypeStruct((NE * HE // 128, 128), jnp.float32),
    )(ef)


def _post_body(sf_ref, ef_ref, nv_ref, wve_ref, bv_ref, out_ref):
    v8 = sf_ref[pl.ds(0, SPAD // 128), :] + sf_ref[pl.ds(SPAD // 128,
                                                         SPAD // 128), :]
    v83 = v8.reshape(SROW, 8, 128)
    nv3 = nv_ref[...].reshape(8, 128, H)
    u = jnp.zeros((SROW, H), jnp.float32)
    denom = jnp.zeros((SROW, 1), jnp.float32)
    for k in range(8):
        blk = v83[:, k, :]
        u = u + jnp.dot(blk, nv3[k], preferred_element_type=jnp.float32,
                        precision=lax.Precision.HIGHEST)
        denom = denom + jnp.sum(blk, axis=1, keepdims=True)
    ef = jnp.sum(ef_ref[...], axis=0)
    efu = jnp.dot(ef, wve_ref[...], preferred_element_type=jnp.float32,
                  precision=lax.Precision.HIGHEST)
    u = u + jnp.concatenate(
        [efu, jnp.zeros((SROW - NA, H), jnp.float32)], axis=0)
    scale = 1.0 / (denom + 1e-16)
    out_ref[...] = u * scale + (denom * scale) * bv_ref[...]


def _post(sf, ef, nv, wve, bv):
    return pl.pallas_call(
        _post_body,
        out_shape=jax.ShapeDtypeStruct((SROW, H), jnp.float32),
    )(sf, ef, nv, wve, bv)


def _sc_body(src_hbm, dst_hbm, ef_hbm, t1_hbm, qe_hbm,
             outs_hbm, outef_hbm,
             src_v, dst_v, ef_v, idx0, idx1, w0, w1, t2d, a2d,
             ef_acc, qe_v, zbuf, s_sh, sem_in, sem_g, sem_s0, sem_s1):
    cid = lax.axis_index("c")
    sid = lax.axis_index("s")
    wid = cid * 16 + sid
    idx2 = (idx0, idx1)
    w2 = (w0, w1)
    sem_s = (sem_s0, sem_s1)

    def _z16(i, c):
        zbuf[pl.ds(i * 16, 16)] = jnp.zeros((16,), jnp.float32)
        return c
    lax.fori_loop(0, ZB // 16, _z16, 0)

    def _zef(i, c):
        ef_acc[pl.ds(i * 16, 16)] = jnp.zeros((16,), jnp.float32)
        return c
    lax.fori_loop(0, NA * HE // 16, _zef, 0)

    sbase = sid * (SPAD // 16)
    for zi in range(SPAD // 16 // ZB):
        pltpu.sync_copy(zbuf, s_sh.at[pl.ds(sbase + zi * ZB, ZB)])

    pltpu.sync_copy(qe_hbm, qe_v)
    plsc.subcore_barrier()

    pend = {0: [], 1: []}
    for ch in range(NCHUNK):
        par = ch % 2
        idx2d = idx2[par]
        w2d = w2[par]
        ebase = wid * EPW + ch * CH

        c1 = pltpu.async_copy(src_hbm.at[pl.ds(ebase, CH)], src_v, sem_in)
        c2 = pltpu.async_copy(dst_hbm.at[pl.ds(ebase, CH)], dst_v, sem_in)
        c3 = pltpu.async_copy(ef_hbm.at[pl.ds(ebase * HE, CH * HE)], ef_v,
                              sem_in)
        for d in pend[par]:
            d.wait()
        pend[par] = []
        c1.wait()
        c2.wait()
        c3.wait()

        def _l1(g, c):
            r = g // 8
            col = (g % 8) * 16
            sv = src_v[pl.ds(g * 16, 16)]
            dv = dst_v[pl.ds(g * 16, 16)]
            idx2d[r, pl.ds(col, 16)] = dv * SROW + sv
            return c
        lax.fori_loop(0, CH // 16, _l1, 0)

        gath = [pltpu.async_copy(t1_hbm.at[idx2d.at[r]], t2d.at[r], sem_g)
                for r in range(NB)]

        def _l2a(g, c):
            r = g // 8
            col = (g % 8) * 16
            dv = dst_v[pl.ds(g * 16, 16)]
            ebase16 = g * 256 + lax.iota(jnp.int32, 16) * 16
            qbase16 = dv * 16
            acc = jnp.zeros((16,), jnp.float32)
            for j in range(HE):
                qj = plsc.load_gather(qe_v, [qbase16 + j])
                ej = plsc.load_gather(ef_v, [ebase16 + j])
                acc = acc + qj * ej
            a2d[r, pl.ds(col, 16)] = acc
            return c
        lax.fori_loop(0, CH // 16, _l2a, 0)

        for d in gath:
            d.wait()

        def _l2b(g, c):
            r = g // 8
            col = (g % 8) * 16
            dv = dst_v[pl.ds(g * 16, 16)]
            acc = a2d[r, pl.ds(col, 16)] + t2d[r, pl.ds(col, 16)]
            gid = ebase + g * 16 + lax.iota(jnp.int32, 16)
            w = jnp.exp(acc)
            w = jnp.where(gid < NE, w, jnp.zeros((16,), jnp.float32))
            w2d[r, pl.ds(col, 16)] = w
            ebase16 = g * 256 + lax.iota(jnp.int32, 16) * 16
            dbase16 = dv * 16
            for j in range(HE):
                ej = plsc.load_gather(ef_v, [ebase16 + j])
                plsc.addupdate_scatter(ef_acc, [dbase16 + j], w * ej)
            return c
        lax.fori_loop(0, CH // 16, _l2b, 0)

        pend[par] = [
            pltpu.async_copy(w2d.at[r], s_sh.at[idx2d.at[r]], sem_s[par],
                             add=True)
            for r in range(NB)
        ]

    for par in (0, 1):
        for d in pend[par]:
            d.wait()

    plsc.subcore_barrier()
    obase = sid * (SPAD // 16)
    pltpu.sync_copy(s_sh.at[pl.ds(obase, SPAD // 16)],
                    outs_hbm.at[cid, pl.ds(obase, SPAD // 16)])
    pltpu.sync_copy(ef_acc, outef_hbm.at[wid])


@functools.partial(
    pl.kernel,
    out_type=(
        jax.ShapeDtypeStruct((2, SPAD), jnp.float32),
        jax.ShapeDtypeStruct((NWORK, NA * HE), jnp.float32),
    ),
    mesh=plsc.VectorSubcoreMesh(core_axis_name="c", subcore_axis_name="s"),
    compiler_params=pltpu.CompilerParams(needs_layout_passes=False),
    scratch_types=(
        pltpu.VMEM((CH,), jnp.int32),
        pltpu.VMEM((CH,), jnp.int32),
        pltpu.VMEM((CH * HE,), jnp.float32),
        pltpu.VMEM((NB, 128), jnp.int32),
        pltpu.VMEM((NB, 128), jnp.int32),
        pltpu.VMEM((NB, 128), jnp.float32),
        pltpu.VMEM((NB, 128), jnp.float32),
        pltpu.VMEM((NB, 128), jnp.float32),
        pltpu.VMEM((NB, 128), jnp.float32),
        pltpu.VMEM((NA * HE,), jnp.float32),
        pltpu.VMEM((NA * HE,), jnp.float32),
        pltpu.VMEM((ZB,), jnp.float32),
        pltpu.VMEM_SHARED((SPAD,), jnp.float32),
        pltpu.SemaphoreType.DMA,
        pltpu.SemaphoreType.DMA,
        pltpu.SemaphoreType.DMA,
        pltpu.SemaphoreType.DMA,
    ),
)
def _sc_edges(src_hbm, dst_hbm, ef_hbm, t1_hbm, qe_hbm, outs_hbm, outef_hbm,
              src_v, dst_v, ef_v, idx0, idx1, w0, w1, t2d, a2d, ef_acc, qe_v,
              zbuf, s_sh, sem_in, sem_g, sem_s0, sem_s1):
    _sc_body(src_hbm, dst_hbm, ef_hbm, t1_hbm, qe_hbm, outs_hbm, outef_hbm,
             src_v, dst_v, ef_v, idx0, idx1, w0, w1, t2d, a2d, ef_acc, qe_v,
             zbuf, s_sh, sem_in, sem_g, sem_s0, sem_s1)


@jax.jit
def kernel(anchor_features, node_features, a2n_edge_index, a2n_edge_features,
           W_q, b_q, W_kv, b_kv):
    src = a2n_edge_index[0].astype(jnp.int32)
    dst = a2n_edge_index[1].astype(jnp.int32)

    def _r(x):
        u = lax.bitcast_convert_type(x, jnp.uint32)
        lsb = (u >> jnp.uint32(16)) & jnp.uint32(1)
        u = (u + jnp.uint32(0x7FFF) + lsb) & jnp.uint32(0xFFFF0000)
        return lax.bitcast_convert_type(u, jnp.float32)

    wkv_r = _r(W_kv)
    wkn = wkv_r[:H, :H]
    wvn = wkv_r[:H, H:]
    wke = wkv_r[H:, :H]
    wve = wkv_r[H:, H:]
    bq2 = b_q.reshape(1, H)
    bv2 = b_kv[H:].reshape(1, H)
    nf1p = jnp.pad(node_features[:NA], ((0, SROW - NA), (0, 0)))
    t1s, qe, nv = _pre(_r(anchor_features), _r(nf1p), _r(W_q),
                       bq2, wkn, wke, wvn)
    t1f = t1s.reshape(NA * SROW)
    qef = qe.reshape(NA * HE)

    pad = NEP - NE
    src_p = jnp.concatenate([src, jnp.zeros((pad,), jnp.int32)])
    dst_p = jnp.concatenate([dst, jnp.zeros((pad,), jnp.int32)])
    ef_p = jnp.pad(_r(a2n_edge_features).reshape(NE * HE // 128, 128),
                   ((0, pad * HE // 128), (0, 0))).reshape(NEP * HE)

    s_parts, ef_parts = _sc_edges(src_p, dst_p, ef_p, t1f, qef)

    sf = s_parts.reshape(2 * SPAD // 128, 128)
    ef3 = ef_parts.reshape(NWORK, NA, HE)
    return _post(sf, ef3, nv, wve, bv2)[:NA]


# speedup vs baseline: 1.0040x; 1.0040x over previous
import functools

import jax
import jax.numpy as jnp
from jax import lax
from jax.experimental import pallas as pl
from jax.experimental.pallas import tpu as pltpu
from jax.experimental.pallas import tpu_sc as plsc

H = 128
HE = 16
NA = 1000
NE = 320000

NWORK = 32
CH = 1024
NCHUNK = 10
EPW = CH * NCHUNK
NEP = NWORK * EPW
NB = CH // 128
SROW = 1024
SPAD = 1 << 20
ZB = 8192


def _pre_body(a_ref, nf_ref, wq_ref, bq_ref, wkn_ref, wke_ref, wvn_ref,
              t1s_ref, qe_ref, nv_ref):
    q = jnp.dot(a_ref[...], wq_ref[...], preferred_element_type=jnp.float32,
                precision=lax.Precision.HIGHEST)
    q = q + bq_ref[...]
    nk = jnp.dot(nf_ref[...], wkn_ref[...], preferred_element_type=jnp.float32,
                precision=lax.Precision.HIGHEST)
    nv_ref[...] = jnp.dot(nf_ref[...], wvn_ref[...],
                          preferred_element_type=jnp.float32,
                precision=lax.Precision.HIGHEST)
    t1 = lax.dot_general(q, nk, (((1,), (1,)), ((), ())),
                         preferred_element_type=jnp.float32,
                precision=lax.Precision.HIGHEST)
    t1s_ref[...] = t1 - jnp.max(t1, axis=1, keepdims=True)
    qe_ref[...] = lax.dot_general(q, wke_ref[...], (((1,), (1,)), ((), ())),
                                  preferred_element_type=jnp.float32,
                precision=lax.Precision.HIGHEST)


def _pre(a, nf1, wq, bq, wkn, wke, wvn):
    return pl.pallas_call(
        _pre_body,
        out_shape=(
            jax.ShapeDtypeStruct((NA, SROW), jnp.float32),
            jax.ShapeDtypeStruct((NA, HE), jnp.float32),
            jax.ShapeDtypeStruct((SROW, H), jnp.float32),
        ),
    )(a, nf1, wq, bq, wkn, wke, wvn)


EFBLK = 1600

def _efr_body(ef_ref, out_ref):
    x = ef_ref[...]
    u = lax.bitcast_convert_type(x, jnp.uint32)
    lsb = (u >> jnp.uint32(16)) & jnp.uint32(1)
    u = (u + jnp.uint32(0x7FFF) + lsb) & jnp.uint32(0xFFFF0000)
    y = lax.bitcast_convert_type(u, jnp.float32)
    out_ref[...] = y.reshape(EFBLK * HE // 128, 128)


def _efr(ef):
    nblk = NE // EFBLK
    return pl.pallas_call(
        _efr_body,
        grid=(nblk,),
        in_specs=[pl.BlockSpec((EFBLK, HE), lambda i: (i, 0))],
        out_specs=pl.BlockSpec((EFBLK * HE // 128, 128), lambda i: (i, 0)),
        out_shape=jax.ShapeDtypeStruct((NE * HE // 128, 128), jnp.float32),
    )(ef)


def _post_body(sf_ref, ef_ref, nv_ref, wve_ref, bv_ref, out_ref):
    v8 = sf_ref[pl.ds(0, SPAD // 128), :] + sf_ref[pl.ds(SPAD // 128,
                                                         SPAD // 128), :]
    v83 = v8.reshape(SROW, 8, 128)
    nv3 = nv_ref[...].reshape(8, 128, H)
    u = jnp.zeros((SROW, H), jnp.float32)
    denom = jnp.zeros((SROW, 1), jnp.float32)
    for k in range(8):
        blk = v83[:, k, :]
        u = u + jnp.dot(blk, nv3[k], preferred_element_type=jnp.float32,
                        precision=lax.Precision.HIGHEST)
        denom = denom + jnp.sum(blk, axis=1, keepdims=True)
    ef = jnp.sum(ef_ref[...], axis=0)
    efu = jnp.dot(ef, wve_ref[...], preferred_element_type=jnp.float32,
                  precision=lax.Precision.HIGHEST)
    u = u + jnp.concatenate(
        [efu, jnp.zeros((SROW - NA, H), jnp.float32)], axis=0)
    scale = 1.0 / (denom + 1e-16)
    out_ref[...] = u * scale + (denom * scale) * bv_ref[...]


def _post(sf, ef, nv, wve, bv):
    return pl.pallas_call(
        _post_body,
        out_shape=jax.ShapeDtypeStruct((SROW, H), jnp.float32),
    )(sf, ef, nv, wve, bv)


def _sc_body(src_hbm, dst_hbm, ef_hbm, t1_hbm, qe_hbm,
             outs_hbm, outef_hbm,
             src_v, dst_v, ef_v, idx0, idx1, w0, w1, t2d, a2d,
             ef_acc, qe_v, zbuf, s_sh, sem_in, sem_g, sem_s0, sem_s1):
    cid = lax.axis_index("c")
    sid = lax.axis_index("s")
    wid = cid * 16 + sid
    idx2 = (idx0, idx1)
    w2 = (w0, w1)
    sem_s = (sem_s0, sem_s1)

    def _z16(i, c):
        zbuf[pl.ds(i * 16, 16)] = jnp.zeros((16,), jnp.float32)
        return c
    lax.fori_loop(0, ZB // 16, _z16, 0)

    def _zef(i, c):
        ef_acc[pl.ds(i * 16, 16)] = jnp.zeros((16,), jnp.float32)
        return c
    lax.fori_loop(0, NA * HE // 16, _zef, 0)

    sbase = sid * (SPAD // 16)
    for zi in range(SPAD // 16 // ZB):
        pltpu.sync_copy(zbuf, s_sh.at[pl.ds(sbase + zi * ZB, ZB)])

    pltpu.sync_copy(qe_hbm, qe_v)
    plsc.subcore_barrier()

    pend = {0: [], 1: []}
    for ch in range(NCHUNK):
        par = ch % 2
        idx2d = idx2[par]
        w2d = w2[par]
        ebase = wid * EPW + ch * CH

        c1 = pltpu.async_copy(src_hbm.at[pl.ds(ebase, CH)], src_v, sem_in)
        c2 = pltpu.async_copy(dst_hbm.at[pl.ds(ebase, CH)], dst_v, sem_in)
        c3 = pltpu.async_copy(ef_hbm.at[pl.ds(ebase * HE, CH * HE)], ef_v,
                              sem_in)
        for d in pend[par]:
            d.wait()
        pend[par] = []
        c1.wait()
        c2.wait()
        c3.wait()

        def _l1(g, c):
            r = g // 8
            col = (g % 8) * 16
            sv = src_v[pl.ds(g * 16, 16)]
            dv = dst_v[pl.ds(g * 16, 16)]
            idx2d[r, pl.ds(col, 16)] = dv * SROW + sv
            return c
        lax.fori_loop(0, CH // 16, _l1, 0)

        gath = [pltpu.async_copy(t1_hbm.at[idx2d.at[r]], t2d.at[r], sem_g)
                for r in range(NB)]

        def _l2a(g, c):
            r = g // 8
            col = (g % 8) * 16
            dv = dst_v[pl.ds(g * 16, 16)]
            ebase16 = g * 256 + lax.iota(jnp.int32, 16) * 16
            qbase16 = dv * 16
            acc = jnp.zeros((16,), jnp.float32)
            for j in range(HE):
                qj = plsc.load_gather(qe_v, [qbase16 + j])
                ej = plsc.load_gather(ef_v, [ebase16 + j])
                acc = acc + qj * ej
            a2d[r, pl.ds(col, 16)] = acc
            return c
        lax.fori_loop(0, CH // 16, _l2a, 0)

        for d in gath:
            d.wait()

        def _l2b(g, c):
            r = g // 8
            col = (g % 8) * 16
            dv = dst_v[pl.ds(g * 16, 16)]
            acc = a2d[r, pl.ds(col, 16)] + t2d[r, pl.ds(col, 16)]
            gid = ebase + g * 16 + lax.iota(jnp.int32, 16)
            w = jnp.exp(acc)
            w = jnp.where(gid < NE, w, jnp.zeros((16,), jnp.float32))
            w2d[r, pl.ds(col, 16)] = w
            ebase16 = g * 256 + lax.iota(jnp.int32, 16) * 16
            dbase16 = dv * 16
            for j in range(HE):
                ej = plsc.load_gather(ef_v, [ebase16 + j])
                plsc.addupdate_scatter(ef_acc, [dbase16 + j], w * ej)
            return c
        lax.fori_loop(0, CH // 16, _l2b, 0)

        pend[par] = [
            pltpu.async_copy(w2d.at[r], s_sh.at[idx2d.at[r]], sem_s[par],
                             add=True)
            for r in range(NB)
        ]

    for par in (0, 1):
        for d in pend[par]:
            d.wait()

    plsc.subcore_barrier()
    obase = sid * (SPAD // 16)
    pltpu.sync_copy(s_sh.at[pl.ds(obase, SPAD // 16)],
                    outs_hbm.at[cid, pl.ds(obase, SPAD // 16)])
    pltpu.sync_copy(ef_acc, outef_hbm.at[wid])


@functools.partial(
    pl.kernel,
    out_type=(
        jax.ShapeDtypeStruct((2, SPAD), jnp.float32),
        jax.ShapeDtypeStruct((NWORK, NA * HE), jnp.float32),
    ),
    mesh=plsc.VectorSubcoreMesh(core_axis_name="c", subcore_axis_name="s"),
    compiler_params=pltpu.CompilerParams(needs_layout_passes=False),
    scratch_types=(
        pltpu.VMEM((CH,), jnp.int32),
        pltpu.VMEM((CH,), jnp.int32),
        pltpu.VMEM((CH * HE,), jnp.float32),
        pltpu.VMEM((NB, 128), jnp.int32),
        pltpu.VMEM((NB, 128), jnp.int32),
        pltpu.VMEM((NB, 128), jnp.float32),
        pltpu.VMEM((NB, 128), jnp.float32),
        pltpu.VMEM((NB, 128), jnp.float32),
        pltpu.VMEM((NB, 128), jnp.float32),
        pltpu.VMEM((NA * HE,), jnp.float32),
        pltpu.VMEM((NA * HE,), jnp.float32),
        pltpu.VMEM((ZB,), jnp.float32),
        pltpu.VMEM_SHARED((SPAD,), jnp.float32),
        pltpu.SemaphoreType.DMA,
        pltpu.SemaphoreType.DMA,
        pltpu.SemaphoreType.DMA,
        pltpu.SemaphoreType.DMA,
    ),
)
def _sc_edges(src_hbm, dst_hbm, ef_hbm, t1_hbm, qe_hbm, outs_hbm, outef_hbm,
              src_v, dst_v, ef_v, idx0, idx1, w0, w1, t2d, a2d, ef_acc, qe_v,
              zbuf, s_sh, sem_in, sem_g, sem_s0, sem_s1):
    _sc_body(src_hbm, dst_hbm, ef_hbm, t1_hbm, qe_hbm, outs_hbm, outef_hbm,
             src_v, dst_v, ef_v, idx0, idx1, w0, w1, t2d, a2d, ef_acc, qe_v,
             zbuf, s_sh, sem_in, sem_g, sem_s0, sem_s1)


@jax.jit
def kernel(anchor_features, node_features, a2n_edge_index, a2n_edge_features,
           W_q, b_q, W_kv, b_kv):
    src = a2n_edge_index[0].astype(jnp.int32)
    dst = a2n_edge_index[1].astype(jnp.int32)

    def _r(x):
        u = lax.bitcast_convert_type(x, jnp.uint32)
        lsb = (u >> jnp.uint32(16)) & jnp.uint32(1)
        u = (u + jnp.uint32(0x7FFF) + lsb) & jnp.uint32(0xFFFF0000)
        return lax.bitcast_convert_type(u, jnp.float32)

    wkv_r = _r(W_kv)
    wkn = wkv_r[:H, :H]
    wvn = wkv_r[:H, H:]
    wke = wkv_r[H:, :H]
    wve = wkv_r[H:, H:]
    bq2 = b_q.reshape(1, H)
    bv2 = b_kv[H:].reshape(1, H)
    nf1p = jnp.pad(node_features[:NA], ((0, SROW - NA), (0, 0)))
    t1s, qe, nv = _pre(_r(anchor_features), _r(nf1p), _r(W_q),
                       bq2, wkn, wke, wvn)
    t1f = t1s.reshape(NA * SROW)
    qef = qe.reshape(NA * HE)

    pad = NEP - NE
    src_p = jnp.concatenate([src, jnp.zeros((pad,), jnp.int32)])
    dst_p = jnp.concatenate([dst, jnp.zeros((pad,), jnp.int32)])
    ef_p = jnp.concatenate(
        [_r(a2n_edge_features).reshape(NE * HE),
         jnp.zeros((pad * HE,), jnp.float32)])

    s_parts, ef_parts = _sc_edges(src_p, dst_p, ef_p, t1f, qef)

    sf = s_parts.reshape(2 * SPAD // 128, 128)
    ef3 = ef_parts.reshape(NWORK, NA, HE)
    return _post(sf, ef3, nv, wve, bv2)[:NA]


# speedup vs baseline: 1.2861x; 1.2809x over previous
import functools

import jax
import jax.numpy as jnp
from jax import lax
from jax.experimental import pallas as pl
from jax.experimental.pallas import tpu as pltpu
from jax.experimental.pallas import tpu_sc as plsc

H = 128
HE = 16
NA = 1000
NE = 320000

NWORK = 32
CH = 1024
NCHUNK = 10
EPW = CH * NCHUNK
NEP = NWORK * EPW
NB = CH // 128
SROW = 1024
SPAD = 1 << 20
ZB = 8192


def _pre_body(a_ref, nf_ref, wq_ref, bq_ref, wkn_ref, wke_ref, wvn_ref,
              t1s_ref, qe_ref, nv_ref):
    q = jnp.dot(a_ref[...], wq_ref[...], preferred_element_type=jnp.float32,
                precision=lax.Precision.HIGHEST)
    q = q + bq_ref[...]
    nk = jnp.dot(nf_ref[...], wkn_ref[...], preferred_element_type=jnp.float32,
                precision=lax.Precision.HIGHEST)
    nv_ref[...] = jnp.dot(nf_ref[...], wvn_ref[...],
                          preferred_element_type=jnp.float32,
                precision=lax.Precision.HIGHEST)
    t1 = lax.dot_general(q, nk, (((1,), (1,)), ((), ())),
                         preferred_element_type=jnp.float32,
                precision=lax.Precision.HIGHEST)
    t1s_ref[...] = t1 - jnp.max(t1, axis=1, keepdims=True)
    qe_ref[...] = lax.dot_general(q, wke_ref[...], (((1,), (1,)), ((), ())),
                                  preferred_element_type=jnp.float32,
                precision=lax.Precision.HIGHEST)


def _pre(a, nf1, wq, bq, wkn, wke, wvn):
    return pl.pallas_call(
        _pre_body,
        out_shape=(
            jax.ShapeDtypeStruct((NA, SROW), jnp.float32),
            jax.ShapeDtypeStruct((NA, HE), jnp.float32),
            jax.ShapeDtypeStruct((SROW, H), jnp.float32),
        ),
    )(a, nf1, wq, bq, wkn, wke, wvn)


EFBLK = 1600

def _efr_body(ef_ref, out_ref):
    x = ef_ref[...]
    u = lax.bitcast_convert_type(x, jnp.uint32)
    lsb = (u >> jnp.uint32(16)) & jnp.uint32(1)
    u = (u + jnp.uint32(0x7FFF) + lsb) & jnp.uint32(0xFFFF0000)
    y = lax.bitcast_convert_type(u, jnp.float32)
    out_ref[...] = y.reshape(EFBLK * HE // 128, 128)


def _efr(ef):
    nblk = NE // EFBLK
    return pl.pallas_call(
        _efr_body,
        grid=(nblk,),
        in_specs=[pl.BlockSpec((EFBLK, HE), lambda i: (i, 0))],
        out_specs=pl.BlockSpec((EFBLK * HE // 128, 128), lambda i: (i, 0)),
        out_shape=jax.ShapeDtypeStruct((NE * HE // 128, 128), jnp.float32),
    )(ef)


def _post_body(sf_ref, ef_ref, nv_ref, wve_ref, bv_ref, out_ref):
    v8 = sf_ref[pl.ds(0, SPAD // 128), :] + sf_ref[pl.ds(SPAD // 128,
                                                         SPAD // 128), :]
    v83 = v8.reshape(SROW, 8, 128)
    nv3 = nv_ref[...].reshape(8, 128, H)
    u = jnp.zeros((SROW, H), jnp.float32)
    denom = jnp.zeros((SROW, 1), jnp.float32)
    for k in range(8):
        blk = v83[:, k, :]
        u = u + jnp.dot(blk, nv3[k], preferred_element_type=jnp.float32,
                        precision=lax.Precision.HIGHEST)
        denom = denom + jnp.sum(blk, axis=1, keepdims=True)
    ef = jnp.sum(ef_ref[...], axis=0)
    efu = jnp.dot(ef, wve_ref[...], preferred_element_type=jnp.float32,
                  precision=lax.Precision.HIGHEST)
    u = u + jnp.concatenate(
        [efu, jnp.zeros((SROW - NA, H), jnp.float32)], axis=0)
    scale = 1.0 / (denom + 1e-16)
    out_ref[...] = u * scale + (denom * scale) * bv_ref[...]


def _post(sf, ef, nv, wve, bv):
    return pl.pallas_call(
        _post_body,
        out_shape=jax.ShapeDtypeStruct((SROW, H), jnp.float32),
    )(sf, ef, nv, wve, bv)


def _sc_body(src_hbm, dst_hbm, ef_hbm, t1_hbm, qe_hbm,
             outs_hbm, outef_hbm,
             src_v, dst_v, ef_v, idx0, idx1, w0, w1, t2d, a2d,
             ef_acc, qe_v, zbuf, s_sh, sem_in, sem_g, sem_s0, sem_s1):
    cid = lax.axis_index("c")
    sid = lax.axis_index("s")
    wid = cid * 16 + sid
    idx2 = (idx0, idx1)
    w2 = (w0, w1)
    sem_s = (sem_s0, sem_s1)

    def _z16(i, c):
        zbuf[pl.ds(i * 16, 16)] = jnp.zeros((16,), jnp.float32)
        return c
    lax.fori_loop(0, ZB // 16, _z16, 0)

    def _zef(i, c):
        ef_acc[pl.ds(i * 16, 16)] = jnp.zeros((16,), jnp.float32)
        return c
    lax.fori_loop(0, NA * HE // 16, _zef, 0)

    sbase = sid * (SPAD // 16)
    for zi in range(SPAD // 16 // ZB):
        pltpu.sync_copy(zbuf, s_sh.at[pl.ds(sbase + zi * ZB, ZB)])

    pltpu.sync_copy(qe_hbm, qe_v)
    plsc.subcore_barrier()

    pend = {0: [], 1: []}
    for ch in range(NCHUNK):
        par = ch % 2
        idx2d = idx2[par]
        w2d = w2[par]
        ebase = wid * EPW + ch * CH

        c1 = pltpu.async_copy(src_hbm.at[pl.ds(ebase, CH)], src_v, sem_in)
        c2 = pltpu.async_copy(dst_hbm.at[pl.ds(ebase, CH)], dst_v, sem_in)
        c3 = pltpu.async_copy(ef_hbm.at[pl.ds(ebase * HE, CH * HE)], ef_v,
                              sem_in)
        for d in pend[par]:
            d.wait()
        pend[par] = []
        c1.wait()
        c2.wait()
        c3.wait()

        def _l1(g, c):
            r = g // 8
            col = (g % 8) * 16
            sv = src_v[pl.ds(g * 16, 16)]
            dv = dst_v[pl.ds(g * 16, 16)]
            idx2d[r, pl.ds(col, 16)] = dv * SROW + sv
            return c
        lax.fori_loop(0, CH // 16, _l1, 0, unroll=4)

        gath = [pltpu.async_copy(t1_hbm.at[idx2d.at[r]], t2d.at[r], sem_g)
                for r in range(NB)]

        def _l2a(g, c):
            r = g // 8
            col = (g % 8) * 16
            dv = dst_v[pl.ds(g * 16, 16)]
            ebase16 = g * 256 + lax.iota(jnp.int32, 16) * 16
            qbase16 = dv * 16
            acc = jnp.zeros((16,), jnp.float32)
            for j in range(HE):
                qj = plsc.load_gather(qe_v, [qbase16 + j])
                ej = plsc.load_gather(ef_v, [ebase16 + j])
                acc = acc + qj * ej
            a2d[r, pl.ds(col, 16)] = acc
            return c
        lax.fori_loop(0, CH // 16, _l2a, 0, unroll=2)

        for d in gath:
            d.wait()

        def _l2b(g, c):
            r = g // 8
            col = (g % 8) * 16
            dv = dst_v[pl.ds(g * 16, 16)]
            acc = a2d[r, pl.ds(col, 16)] + t2d[r, pl.ds(col, 16)]
            gid = ebase + g * 16 + lax.iota(jnp.int32, 16)
            w = jnp.exp(acc)
            w = jnp.where(gid < NE, w, jnp.zeros((16,), jnp.float32))
            w2d[r, pl.ds(col, 16)] = w
            for lane in range(16):
                off = dv[lane] * 16
                row = ef_v[pl.ds(g * 256 + lane * 16, 16)]
                ef_acc[pl.ds(off, 16)] = (ef_acc[pl.ds(off, 16)]
                                          + w[lane] * row)
            return c
        lax.fori_loop(0, CH // 16, _l2b, 0)

        pend[par] = [
            pltpu.async_copy(w2d.at[r], s_sh.at[idx2d.at[r]], sem_s[par],
                             add=True)
            for r in range(NB)
        ]

    for par in (0, 1):
        for d in pend[par]:
            d.wait()

    plsc.subcore_barrier()
    obase = sid * (SPAD // 16)
    pltpu.sync_copy(s_sh.at[pl.ds(obase, SPAD // 16)],
                    outs_hbm.at[cid, pl.ds(obase, SPAD // 16)])
    pltpu.sync_copy(ef_acc, outef_hbm.at[wid])


@functools.partial(
    pl.kernel,
    out_type=(
        jax.ShapeDtypeStruct((2, SPAD), jnp.float32),
        jax.ShapeDtypeStruct((NWORK, NA * HE), jnp.float32),
    ),
    mesh=plsc.VectorSubcoreMesh(core_axis_name="c", subcore_axis_name="s"),
    compiler_params=pltpu.CompilerParams(needs_layout_passes=False),
    scratch_types=(
        pltpu.VMEM((CH,), jnp.int32),
        pltpu.VMEM((CH,), jnp.int32),
        pltpu.VMEM((CH * HE,), jnp.float32),
        pltpu.VMEM((NB, 128), jnp.int32),
        pltpu.VMEM((NB, 128), jnp.int32),
        pltpu.VMEM((NB, 128), jnp.float32),
        pltpu.VMEM((NB, 128), jnp.float32),
        pltpu.VMEM((NB, 128), jnp.float32),
        pltpu.VMEM((NB, 128), jnp.float32),
        pltpu.VMEM((NA * HE,), jnp.float32),
        pltpu.VMEM((NA * HE,), jnp.float32),
        pltpu.VMEM((ZB,), jnp.float32),
        pltpu.VMEM_SHARED((SPAD,), jnp.float32),
        pltpu.SemaphoreType.DMA,
        pltpu.SemaphoreType.DMA,
        pltpu.SemaphoreType.DMA,
        pltpu.SemaphoreType.DMA,
    ),
)
def _sc_edges(src_hbm, dst_hbm, ef_hbm, t1_hbm, qe_hbm, outs_hbm, outef_hbm,
              src_v, dst_v, ef_v, idx0, idx1, w0, w1, t2d, a2d, ef_acc, qe_v,
              zbuf, s_sh, sem_in, sem_g, sem_s0, sem_s1):
    _sc_body(src_hbm, dst_hbm, ef_hbm, t1_hbm, qe_hbm, outs_hbm, outef_hbm,
             src_v, dst_v, ef_v, idx0, idx1, w0, w1, t2d, a2d, ef_acc, qe_v,
             zbuf, s_sh, sem_in, sem_g, sem_s0, sem_s1)


@jax.jit
def kernel(anchor_features, node_features, a2n_edge_index, a2n_edge_features,
           W_q, b_q, W_kv, b_kv):
    src = a2n_edge_index[0].astype(jnp.int32)
    dst = a2n_edge_index[1].astype(jnp.int32)

    def _r(x):
        u = lax.bitcast_convert_type(x, jnp.uint32)
        lsb = (u >> jnp.uint32(16)) & jnp.uint32(1)
        u = (u + jnp.uint32(0x7FFF) + lsb) & jnp.uint32(0xFFFF0000)
        return lax.bitcast_convert_type(u, jnp.float32)

    wkv_r = _r(W_kv)
    wkn = wkv_r[:H, :H]
    wvn = wkv_r[:H, H:]
    wke = wkv_r[H:, :H]
    wve = wkv_r[H:, H:]
    bq2 = b_q.reshape(1, H)
    bv2 = b_kv[H:].reshape(1, H)
    nf1p = jnp.pad(node_features[:NA], ((0, SROW - NA), (0, 0)))
    t1s, qe, nv = _pre(_r(anchor_features), _r(nf1p), _r(W_q),
                       bq2, wkn, wke, wvn)
    t1f = t1s.reshape(NA * SROW)
    qef = qe.reshape(NA * HE)

    pad = NEP - NE
    src_p = jnp.concatenate([src, jnp.zeros((pad,), jnp.int32)])
    dst_p = jnp.concatenate([dst, jnp.zeros((pad,), jnp.int32)])
    ef_p = jnp.concatenate(
        [_r(a2n_edge_features).reshape(NE * HE),
         jnp.zeros((pad * HE,), jnp.float32)])

    s_parts, ef_parts = _sc_edges(src_p, dst_p, ef_p, t1f, qef)

    sf = s_parts.reshape(2 * SPAD // 128, 128)
    ef3 = ef_parts.reshape(NWORK, NA, HE)
    return _post(sf, ef3, nv, wve, bv2)[:NA]


# speedup vs baseline: 1.2863x; 1.0002x over previous
import functools

import jax
import jax.numpy as jnp
from jax import lax
from jax.experimental import pallas as pl
from jax.experimental.pallas import tpu as pltpu
from jax.experimental.pallas import tpu_sc as plsc

H = 128
HE = 16
NA = 1000
NE = 320000

NWORK = 32
CH = 1024
NCHUNK = 10
EPW = CH * NCHUNK
NEP = NWORK * EPW
NB = CH // 128
SROW = 1024
SPAD = 1 << 20
ZB = 8192


def _pre_body(a_ref, nf_ref, wq_ref, bq_ref, wkn_ref, wke_ref, wvn_ref,
              t1s_ref, qe_ref, nv_ref):
    q = jnp.dot(a_ref[...], wq_ref[...], preferred_element_type=jnp.float32,
                precision=lax.Precision.HIGHEST)
    q = q + bq_ref[...]
    nk = jnp.dot(nf_ref[...], wkn_ref[...], preferred_element_type=jnp.float32,
                precision=lax.Precision.HIGHEST)
    nv_ref[...] = jnp.dot(nf_ref[...], wvn_ref[...],
                          preferred_element_type=jnp.float32,
                precision=lax.Precision.HIGHEST)
    t1 = lax.dot_general(q, nk, (((1,), (1,)), ((), ())),
                         preferred_element_type=jnp.float32,
                precision=lax.Precision.HIGHEST)
    t1s_ref[...] = t1 - jnp.max(t1, axis=1, keepdims=True)
    qe_ref[...] = lax.dot_general(q, wke_ref[...], (((1,), (1,)), ((), ())),
                                  preferred_element_type=jnp.float32,
                precision=lax.Precision.HIGHEST)


def _pre(a, nf1, wq, bq, wkn, wke, wvn):
    return pl.pallas_call(
        _pre_body,
        out_shape=(
            jax.ShapeDtypeStruct((NA, SROW), jnp.float32),
            jax.ShapeDtypeStruct((NA, HE), jnp.float32),
            jax.ShapeDtypeStruct((SROW, H), jnp.float32),
        ),
    )(a, nf1, wq, bq, wkn, wke, wvn)


EFBLK = 1600

def _efr_body(ef_ref, out_ref):
    x = ef_ref[...]
    u = lax.bitcast_convert_type(x, jnp.uint32)
    lsb = (u >> jnp.uint32(16)) & jnp.uint32(1)
    u = (u + jnp.uint32(0x7FFF) + lsb) & jnp.uint32(0xFFFF0000)
    y = lax.bitcast_convert_type(u, jnp.float32)
    out_ref[...] = y.reshape(EFBLK * HE // 128, 128)


def _efr(ef):
    nblk = NE // EFBLK
    return pl.pallas_call(
        _efr_body,
        grid=(nblk,),
        in_specs=[pl.BlockSpec((EFBLK, HE), lambda i: (i, 0))],
        out_specs=pl.BlockSpec((EFBLK * HE // 128, 128), lambda i: (i, 0)),
        out_shape=jax.ShapeDtypeStruct((NE * HE // 128, 128), jnp.float32),
    )(ef)


def _post_body(sf_ref, ef_ref, nv_ref, wve_ref, bv_ref, out_ref):
    v8 = sf_ref[pl.ds(0, SPAD // 128), :] + sf_ref[pl.ds(SPAD // 128,
                                                         SPAD // 128), :]
    v83 = v8.reshape(SROW, 8, 128)
    nv3 = nv_ref[...].reshape(8, 128, H)
    u = jnp.zeros((SROW, H), jnp.float32)
    denom = jnp.zeros((SROW, 1), jnp.float32)
    for k in range(8):
        blk = v83[:, k, :]
        u = u + jnp.dot(blk, nv3[k], preferred_element_type=jnp.float32,
                        precision=lax.Precision.HIGHEST)
        denom = denom + jnp.sum(blk, axis=1, keepdims=True)
    ef = jnp.sum(ef_ref[...], axis=0)
    efu = jnp.dot(ef, wve_ref[...], preferred_element_type=jnp.float32,
                  precision=lax.Precision.HIGHEST)
    u = u + jnp.concatenate(
        [efu, jnp.zeros((SROW - NA, H), jnp.float32)], axis=0)
    scale = 1.0 / (denom + 1e-16)
    out_ref[...] = u * scale + (denom * scale) * bv_ref[...]


def _post(sf, ef, nv, wve, bv):
    return pl.pallas_call(
        _post_body,
        out_shape=jax.ShapeDtypeStruct((SROW, H), jnp.float32),
    )(sf, ef, nv, wve, bv)


def _sc_body(src_hbm, dst_hbm, ef_hbm, t1_hbm, qe_hbm,
             outs_hbm, outef_hbm,
             src_v, dst_v, ef_v, idx0, idx1, w0, w1, t2d, a2d,
             ef_acc, qe_v, zbuf, s_sh, sem_in, sem_g, sem_s0, sem_s1):
    cid = lax.axis_index("c")
    sid = lax.axis_index("s")
    wid = cid * 16 + sid
    idx2 = (idx0, idx1)
    w2 = (w0, w1)
    sem_s = (sem_s0, sem_s1)

    def _z16(i, c):
        zbuf[pl.ds(i * 16, 16)] = jnp.zeros((16,), jnp.float32)
        return c
    lax.fori_loop(0, ZB // 16, _z16, 0)

    def _zef(i, c):
        ef_acc[pl.ds(i * 16, 16)] = jnp.zeros((16,), jnp.float32)
        return c
    lax.fori_loop(0, NA * HE // 16, _zef, 0)

    sbase = sid * (SPAD // 16)
    for zi in range(SPAD // 16 // ZB):
        pltpu.sync_copy(zbuf, s_sh.at[pl.ds(sbase + zi * ZB, ZB)])

    pltpu.sync_copy(qe_hbm, qe_v)
    plsc.subcore_barrier()

    pend = {0: [], 1: []}
    for ch in range(NCHUNK):
        par = ch % 2
        idx2d = idx2[par]
        w2d = w2[par]
        ebase = wid * EPW + ch * CH

        c1 = pltpu.async_copy(src_hbm.at[pl.ds(ebase, CH)], src_v, sem_in)
        c2 = pltpu.async_copy(dst_hbm.at[pl.ds(ebase, CH)], dst_v, sem_in)
        c3 = pltpu.async_copy(ef_hbm.at[pl.ds(ebase * HE, CH * HE)], ef_v,
                              sem_in)
        for d in pend[par]:
            d.wait()
        pend[par] = []
        c1.wait()
        c2.wait()
        c3.wait()

        def _l1(g, c):
            r = g // 8
            col = (g % 8) * 16
            sv = src_v[pl.ds(g * 16, 16)]
            dv = dst_v[pl.ds(g * 16, 16)]
            idx2d[r, pl.ds(col, 16)] = dv * SROW + sv
            return c
        lax.fori_loop(0, CH // 16, _l1, 0, unroll=4)

        gath = [pltpu.async_copy(t1_hbm.at[idx2d.at[r]], t2d.at[r], sem_g)
                for r in range(NB)]

        def _l2a(g, c):
            r = g // 8
            col = (g % 8) * 16
            dv = dst_v[pl.ds(g * 16, 16)]
            ebase16 = g * 256 + lax.iota(jnp.int32, 16) * 16
            qbase16 = dv * 16
            acc = jnp.zeros((16,), jnp.float32)
            for j in range(HE):
                qj = plsc.load_gather(qe_v, [qbase16 + j])
                ej = plsc.load_gather(ef_v, [ebase16 + j])
                acc = acc + qj * ej
            a2d[r, pl.ds(col, 16)] = acc
            return c
        lax.fori_loop(0, CH // 16, _l2a, 0, unroll=2)

        for d in gath:
            d.wait()

        def _l2b(g, c):
            r = g // 8
            col = (g % 8) * 16
            dv = dst_v[pl.ds(g * 16, 16)]
            acc = a2d[r, pl.ds(col, 16)] + t2d[r, pl.ds(col, 16)]
            gid = ebase + g * 16 + lax.iota(jnp.int32, 16)
            w = jnp.exp(acc)
            w = jnp.where(gid < NE, w, jnp.zeros((16,), jnp.float32))
            w2d[r, pl.ds(col, 16)] = w
            for lane in range(16):
                off = dv[lane] * 16
                row = ef_v[pl.ds(g * 256 + lane * 16, 16)]
                ef_acc[pl.ds(off, 16)] = (ef_acc[pl.ds(off, 16)]
                                          + w[lane] * row)
            return c
        lax.fori_loop(0, CH // 16, _l2b, 0, unroll=2)

        pend[par] = [
            pltpu.async_copy(w2d.at[r], s_sh.at[idx2d.at[r]], sem_s[par],
                             add=True)
            for r in range(NB)
        ]

    for par in (0, 1):
        for d in pend[par]:
            d.wait()

    plsc.subcore_barrier()
    obase = sid * (SPAD // 16)
    pltpu.sync_copy(s_sh.at[pl.ds(obase, SPAD // 16)],
                    outs_hbm.at[cid, pl.ds(obase, SPAD // 16)])
    pltpu.sync_copy(ef_acc, outef_hbm.at[wid])


@functools.partial(
    pl.kernel,
    out_type=(
        jax.ShapeDtypeStruct((2, SPAD), jnp.float32),
        jax.ShapeDtypeStruct((NWORK, NA * HE), jnp.float32),
    ),
    mesh=plsc.VectorSubcoreMesh(core_axis_name="c", subcore_axis_name="s"),
    compiler_params=pltpu.CompilerParams(needs_layout_passes=False),
    scratch_types=(
        pltpu.VMEM((CH,), jnp.int32),
        pltpu.VMEM((CH,), jnp.int32),
        pltpu.VMEM((CH * HE,), jnp.float32),
        pltpu.VMEM((NB, 128), jnp.int32),
        pltpu.VMEM((NB, 128), jnp.int32),
        pltpu.VMEM((NB, 128), jnp.float32),
        pltpu.VMEM((NB, 128), jnp.float32),
        pltpu.VMEM((NB, 128), jnp.float32),
        pltpu.VMEM((NB, 128), jnp.float32),
        pltpu.VMEM((NA * HE,), jnp.float32),
        pltpu.VMEM((NA * HE,), jnp.float32),
        pltpu.VMEM((ZB,), jnp.float32),
        pltpu.VMEM_SHARED((SPAD,), jnp.float32),
        pltpu.SemaphoreType.DMA,
        pltpu.SemaphoreType.DMA,
        pltpu.SemaphoreType.DMA,
        pltpu.SemaphoreType.DMA,
    ),
)
def _sc_edges(src_hbm, dst_hbm, ef_hbm, t1_hbm, qe_hbm, outs_hbm, outef_hbm,
              src_v, dst_v, ef_v, idx0, idx1, w0, w1, t2d, a2d, ef_acc, qe_v,
              zbuf, s_sh, sem_in, sem_g, sem_s0, sem_s1):
    _sc_body(src_hbm, dst_hbm, ef_hbm, t1_hbm, qe_hbm, outs_hbm, outef_hbm,
             src_v, dst_v, ef_v, idx0, idx1, w0, w1, t2d, a2d, ef_acc, qe_v,
             zbuf, s_sh, sem_in, sem_g, sem_s0, sem_s1)


@jax.jit
def kernel(anchor_features, node_features, a2n_edge_index, a2n_edge_features,
           W_q, b_q, W_kv, b_kv):
    src = a2n_edge_index[0].astype(jnp.int32)
    dst = a2n_edge_index[1].astype(jnp.int32)

    def _r(x):
        u = lax.bitcast_convert_type(x, jnp.uint32)
        lsb = (u >> jnp.uint32(16)) & jnp.uint32(1)
        u = (u + jnp.uint32(0x7FFF) + lsb) & jnp.uint32(0xFFFF0000)
        return lax.bitcast_convert_type(u, jnp.float32)

    wkv_r = _r(W_kv)
    wkn = wkv_r[:H, :H]
    wvn = wkv_r[:H, H:]
    wke = wkv_r[H:, :H]
    wve = wkv_r[H:, H:]
    bq2 = b_q.reshape(1, H)
    bv2 = b_kv[H:].reshape(1, H)
    nf1p = jnp.pad(node_features[:NA], ((0, SROW - NA), (0, 0)))
    t1s, qe, nv = _pre(_r(anchor_features), _r(nf1p), _r(W_q),
                       bq2, wkn, wke, wvn)
    t1f = t1s.reshape(NA * SROW)
    qef = qe.reshape(NA * HE)

    pad = NEP - NE
    src_p = jnp.concatenate([src, jnp.zeros((pad,), jnp.int32)])
    dst_p = jnp.concatenate([dst, jnp.zeros((pad,), jnp.int32)])
    ef_p = jnp.concatenate(
        [_r(a2n_edge_features).reshape(NE * HE),
         jnp.zeros((pad * HE,), jnp.float32)])

    s_parts, ef_parts = _sc_edges(src_p, dst_p, ef_p, t1f, qef)

    sf = s_parts.reshape(2 * SPAD // 128, 128)
    ef3 = ef_parts.reshape(NWORK, NA, HE)
    return _post(sf, ef3, nv, wve, bv2)[:NA]
